# Initial kernel scaffold; baseline (speedup 1.0000x reference)
import functools

import jax
import jax.numpy as jnp
from jax import lax
from jax.experimental import pallas as pl
from jax.experimental.pallas import tpu as pltpu
from jax.experimental.pallas import tpu_sc as plsc

N = 10000
E = 320000
D_IN = 128
D_H = 256
HALF = 128
G = 64
C = 40

BM = 1000
NB = N // BM
NT = 16
CH = 80
EPT = E // NT
F32 = jnp.float32

_SC_MESH = plsc.VectorSubcoreMesh(core_axis_name="c", subcore_axis_name="s")
_HI = lax.Precision.HIGHEST


def _transform_body(x_ref, w_ref, b_ref, o_ref):
    o_ref[...] = (
        jnp.dot(x_ref[...], w_ref[...], preferred_element_type=F32,
                precision=_HI)
        + b_ref[...]
    )


def _tc_transform(pos, w, b):
    return pl.pallas_call(
        _transform_body,
        grid=(NB,),
        in_specs=[
            pl.BlockSpec((BM, D_IN), lambda i: (i, 0)),
            pl.BlockSpec((D_IN, D_IN), lambda i: (0, 0)),
            pl.BlockSpec((1, D_IN), lambda i: (0, 0)),
        ],
        out_specs=pl.BlockSpec((BM, D_IN), lambda i: (i, 0)),
        out_shape=jax.ShapeDtypeStruct((N, D_IN), F32),
    )(pos, w, b.reshape(1, D_IN))


def _pre_body_scaled(x_ref, w_ref, dinv_ref, o_ref):
    o_ref[...] = dinv_ref[...] * jnp.dot(
        x_ref[...], w_ref[...], preferred_element_type=F32, precision=_HI)


def _pre_body(x_ref, w_ref, o_ref):
    o_ref[...] = jnp.dot(
        x_ref[...], w_ref[...], preferred_element_type=F32, precision=_HI)


def _tc_pre(x, w, dinv=None):
    d = x.shape[1]
    if dinv is not None:
        body = _pre_body_scaled
        extra_specs = [pl.BlockSpec((BM, 1), lambda h, i: (i, 0))]
        extra_args = (dinv,)
    else:
        body = _pre_body
        extra_specs = []
        extra_args = ()
    return pl.pallas_call(
        body,
        grid=(2, NB),
        in_specs=[
            pl.BlockSpec((BM, d), lambda h, i: (i, 0)),
            pl.BlockSpec((d, HALF), lambda h, i: (0, h)),
        ] + extra_specs,
        out_specs=pl.BlockSpec((BM, HALF), lambda h, i: (h * NB + i, 0)),
        out_shape=jax.ShapeDtypeStruct((2 * N, HALF), F32),
    )(x, w, *extra_args)


def _stats_body(acc_ref, dinv_ref, b_ref, o_ref):
    i = pl.program_id(1)

    @pl.when(i == 0)
    def _():
        o_ref[...] = jnp.zeros_like(o_ref)

    z = dinv_ref[...] * acc_ref[...] + b_ref[...]
    s1 = jnp.sum(z, axis=0, keepdims=True)
    s2 = jnp.sum(z * z, axis=0, keepdims=True)
    o_ref[...] += jnp.stack([s1, s2], axis=0)


def _tc_stats(acc, dinv, b2):
    return pl.pallas_call(
        _stats_body,
        grid=(2, NB),
        in_specs=[
            pl.BlockSpec((BM, HALF), lambda h, i: (h * NB + i, 0)),
            pl.BlockSpec((BM, 1), lambda h, i: (i, 0)),
            pl.BlockSpec((1, HALF), lambda h, i: (h, 0)),
        ],
        out_specs=pl.BlockSpec((2, 1, HALF), lambda h, i: (0, h, 0)),
        out_shape=jax.ShapeDtypeStruct((2, 2, HALF), F32),
    )(acc, dinv, b2)


def _apply_body(acc_ref, dinv_ref, b_ref, sc_ref, sh_ref, o_ref, *,
                fold_dinv):
    z = dinv_ref[...] * acc_ref[...] + b_ref[...]
    t = z * sc_ref[...] + sh_ref[...]
    t = jnp.maximum(t, 0.0)
    if fold_dinv:
        t = t * dinv_ref[...]
    o_ref[...] = t


def _tc_apply(acc, dinv, b2, scale2, shift2, fold_dinv):
    return pl.pallas_call(
        functools.partial(_apply_body, fold_dinv=fold_dinv),
        grid=(2, NB),
        in_specs=[
            pl.BlockSpec((BM, HALF), lambda h, i: (h * NB + i, 0)),
            pl.BlockSpec((BM, 1), lambda h, i: (i, 0)),
            pl.BlockSpec((1, HALF), lambda h, i: (h, 0)),
            pl.BlockSpec((1, HALF), lambda h, i: (h, 0)),
            pl.BlockSpec((1, HALF), lambda h, i: (h, 0)),
        ],
        out_specs=pl.BlockSpec((BM, HALF), lambda h, i: (i, h)),
        out_shape=jax.ShapeDtypeStruct((N, D_H), F32),
    )(acc, dinv, b2, scale2, shift2)


def _pool_body(x_ref, batch_ref, w0_ref, b0_ref, w1_ref, b1_ref, o_ref,
               acc_s, cnt_s):
    i = pl.program_id(0)

    @pl.when(i == 0)
    def _():
        acc_s[...] = jnp.zeros_like(acc_s)
        cnt_s[...] = jnp.zeros_like(cnt_s)

    iota = lax.broadcasted_iota(jnp.int32, (G, BM), 0)
    onehot = (iota == batch_ref[...]).astype(F32)
    acc_s[...] += jnp.dot(onehot, x_ref[...], preferred_element_type=F32,
                          precision=_HI)
    cnt_s[...] += jnp.sum(onehot, axis=1, keepdims=True)

    @pl.when(i == NB - 1)
    def _():
        h = acc_s[...] / jnp.maximum(cnt_s[...], 1.0)
        h1 = jnp.maximum(
            jnp.dot(h, w0_ref[...], preferred_element_type=F32,
                    precision=_HI) + b0_ref[...], 0.0)
        o_ref[...] = jnp.dot(h1, w1_ref[...], preferred_element_type=F32,
                             precision=_HI) + b1_ref[...]


def _tc_pool(x, batch2d, w0, b0, w1, b1):
    return pl.pallas_call(
        _pool_body,
        grid=(NB,),
        in_specs=[
            pl.BlockSpec((BM, D_H), lambda i: (i, 0)),
            pl.BlockSpec((1, BM), lambda i: (0, i)),
            pl.BlockSpec((D_H, D_H), lambda i: (0, 0)),
            pl.BlockSpec((1, D_H), lambda i: (0, 0)),
            pl.BlockSpec((D_H, C), lambda i: (0, 0)),
            pl.BlockSpec((1, C), lambda i: (0, 0)),
        ],
        out_specs=pl.BlockSpec((G, C), lambda i: (0, 0)),
        out_shape=jax.ShapeDtypeStruct((G, C), F32),
        scratch_shapes=[
            pltpu.VMEM((G, D_H), F32),
            pltpu.VMEM((G, 1), F32),
        ],
    )(x, batch2d, w0, b0.reshape(1, D_H), w1, b1.reshape(1, C))


_DEG_EPS = E // 2 // NT
_DEG_NCH = _DEG_EPS // CH


def _deg_body(col_hbm, out_hbm, colbuf, ones_v, zbuf, acc_shared, sem):
    c = lax.axis_index("c")
    s = lax.axis_index("s")

    for k in range(CH // 16):
        ones_v[pl.ds(k * 16, 16)] = jnp.ones((16,), F32)
    for k in range(640 // 16):
        zbuf[pl.ds(k * 16, 16)] = jnp.zeros((16,), F32)

    start = s * 624
    pltpu.sync_copy(zbuf.at[pl.ds(0, 624)], acc_shared.at[pl.ds(start, 624)])

    @pl.when(s == NT - 1)
    def _():
        pltpu.sync_copy(zbuf.at[pl.ds(0, 16)], acc_shared.at[pl.ds(9984, 16)])

    plsc.subcore_barrier()

    base = c * (E // 2) + s * _DEG_EPS

    def body(j, carry):
        off = base + j * CH
        pltpu.sync_copy(col_hbm.at[pl.ds(off, CH)], colbuf)
        pltpu.sync_copy(ones_v, acc_shared.at[colbuf], add=True)
        return carry

    lax.fori_loop(0, _DEG_NCH, body, 0, unroll=False)

    plsc.subcore_barrier()

    pltpu.sync_copy(acc_shared.at[pl.ds(start, 624)],
                    out_hbm.at[c].at[pl.ds(start, 624)])

    @pl.when(s == NT - 1)
    def _():
        pltpu.sync_copy(acc_shared.at[pl.ds(9984, 16)],
                        out_hbm.at[c].at[pl.ds(9984, 16)])


_sc_degree = pl.kernel(
    _deg_body,
    out_type=jax.ShapeDtypeStruct((2, N), F32),
    mesh=_SC_MESH,
    scratch_types=[
        pltpu.VMEM((CH,), jnp.int32),
        pltpu.VMEM((CH,), F32),
        pltpu.VMEM((640,), F32),
        pltpu.VMEM_SHARED((N,), F32),
        pltpu.SemaphoreType.DMA,
    ],
)


_ROWS_PT = N // NT
_PROP_NCH = EPT // CH


def _prop_body(ys_hbm, row_hbm, col_hbm, out_hbm, ridx_raw, ridx, cidx,
               rows, acc_shared, sem):
    c = lax.axis_index("c")
    s = lax.axis_index("s")
    c_n = c * N

    init = s * _ROWS_PT
    pltpu.sync_copy(ys_hbm.at[pl.ds(c_n + init, _ROWS_PT)],
                    acc_shared.at[pl.ds(init, _ROWS_PT)])
    plsc.subcore_barrier()

    base = s * EPT

    def body(j, carry):
        off = base + j * CH
        pltpu.sync_copy(row_hbm.at[pl.ds(off, CH)], ridx_raw)
        pltpu.sync_copy(col_hbm.at[pl.ds(off, CH)], cidx)
        for k in range(CH // 16):
            ridx[pl.ds(k * 16, 16)] = ridx_raw[pl.ds(k * 16, 16)] + c_n
        pltpu.async_copy(ys_hbm.at[ridx], rows, sem).wait()
        pltpu.sync_copy(rows, acc_shared.at[cidx], add=True)
        return carry

    lax.fori_loop(0, _PROP_NCH, body, 0, unroll=False)

    plsc.subcore_barrier()

    pltpu.sync_copy(acc_shared.at[pl.ds(init, _ROWS_PT)],
                    out_hbm.at[pl.ds(c_n + init, _ROWS_PT)])


_sc_propagate = pl.kernel(
    _prop_body,
    out_type=jax.ShapeDtypeStruct((2 * N, HALF), F32),
    mesh=_SC_MESH,
    scratch_types=[
        pltpu.VMEM((CH,), jnp.int32),
        pltpu.VMEM((CH,), jnp.int32),
        pltpu.VMEM((CH,), jnp.int32),
        pltpu.VMEM((CH, HALF), F32),
        pltpu.VMEM_SHARED((N, HALF), F32),
        pltpu.SemaphoreType.DMA,
    ],
)


def kernel(pos, edge_index, batch, transform_W, transform_b,
           conv_W0, conv_b0, bn_g0, bn_b0,
           conv_W1, conv_b1, bn_g1, bn_b1,
           conv_W2, conv_b2, bn_g2, bn_b2,
           lin_W0, lin_b0, lin_W1, lin_b1):
    row = edge_index[0]
    col = edge_index[1]

    deg_parts = _sc_degree(col)
    x0 = _tc_transform(pos, transform_W, transform_b)

    deg = deg_parts[0] + deg_parts[1] + 1.0
    dinv = (deg ** -0.5).reshape(N, 1)

    layers = [(conv_W0, conv_b0, bn_g0, bn_b0),
              (conv_W1, conv_b1, bn_g1, bn_b1),
              (conv_W2, conv_b2, bn_g2, bn_b2)]

    x = x0
    out_feat = None
    for li, (w, b, g, be) in enumerate(layers):
        ys = _tc_pre(x, w, dinv if li == 0 else None)
        acc = _sc_propagate(ys, row, col)
        b2 = b.reshape(2, HALF)
        sums = _tc_stats(acc, dinv, b2)
        mu = sums[0] / N
        var = sums[1] / N - mu * mu
        scale2 = g.reshape(2, HALF) * lax.rsqrt(var + 1e-5)
        shift2 = be.reshape(2, HALF) - mu * scale2
        x = _tc_apply(acc, dinv, b2, scale2, shift2, fold_dinv=(li < 2))
        if li == 2:
            out_feat = x

    h = _tc_pool(out_feat, batch.reshape(1, N), lin_W0, lin_b0,
                 lin_W1, lin_b1)
    return (h, out_feat)


# speedup vs baseline: 9.2057x; 9.2057x over previous
import functools

import jax
import jax.numpy as jnp
from jax import lax
from jax.experimental import pallas as pl
from jax.experimental.pallas import tpu as pltpu
from jax.experimental.pallas import tpu_sc as plsc

N = 10000
E = 320000
D_IN = 128
D_H = 256
HALF = 128
G = 64
C = 40

BM = 1000
NB = N // BM
NT = 16
CH = 128
EPT = E // NT
F32 = jnp.float32

_SC_MESH = plsc.VectorSubcoreMesh(core_axis_name="c", subcore_axis_name="s")
_HI = lax.Precision.HIGHEST


def _transform_body(x_ref, w_ref, b_ref, o_ref):
    o_ref[...] = (
        jnp.dot(x_ref[...], w_ref[...], preferred_element_type=F32,
                precision=_HI)
        + b_ref[...]
    )


def _tc_transform(pos, w, b):
    return pl.pallas_call(
        _transform_body,
        grid=(NB,),
        in_specs=[
            pl.BlockSpec((BM, D_IN), lambda i: (i, 0)),
            pl.BlockSpec((D_IN, D_IN), lambda i: (0, 0)),
            pl.BlockSpec((1, D_IN), lambda i: (0, 0)),
        ],
        out_specs=pl.BlockSpec((BM, D_IN), lambda i: (i, 0)),
        out_shape=jax.ShapeDtypeStruct((N, D_IN), F32),
    )(pos, w, b.reshape(1, D_IN))


def _pre_body_scaled(x_ref, w_ref, dinv_ref, o_ref):
    o_ref[...] = dinv_ref[...] * jnp.dot(
        x_ref[...], w_ref[...], preferred_element_type=F32, precision=_HI)


def _pre_body(x_ref, w_ref, o_ref):
    o_ref[...] = jnp.dot(
        x_ref[...], w_ref[...], preferred_element_type=F32, precision=_HI)


def _tc_pre(x, w, dinv=None):
    d = x.shape[1]
    if dinv is not None:
        body = _pre_body_scaled
        extra_specs = [pl.BlockSpec((BM, 1), lambda h, i: (i, 0))]
        extra_args = (dinv,)
    else:
        body = _pre_body
        extra_specs = []
        extra_args = ()
    return pl.pallas_call(
        body,
        grid=(2, NB),
        in_specs=[
            pl.BlockSpec((BM, d), lambda h, i: (i, 0)),
            pl.BlockSpec((d, HALF), lambda h, i: (0, h)),
        ] + extra_specs,
        out_specs=pl.BlockSpec((BM, HALF), lambda h, i: (h * NB + i, 0)),
        out_shape=jax.ShapeDtypeStruct((2 * N, HALF), F32),
    )(x, w, *extra_args)


def _stats_body(acc_ref, dinv_ref, b_ref, o_ref):
    i = pl.program_id(1)

    @pl.when(i == 0)
    def _():
        o_ref[...] = jnp.zeros_like(o_ref)

    z = dinv_ref[...] * acc_ref[...] + b_ref[0]
    s1 = jnp.sum(z, axis=0, keepdims=True)
    s2 = jnp.sum(z * z, axis=0, keepdims=True)
    o_ref[...] += jnp.stack([s1, s2], axis=1)


def _tc_stats(acc, dinv, b2):
    return pl.pallas_call(
        _stats_body,
        grid=(2, NB),
        in_specs=[
            pl.BlockSpec((BM, HALF), lambda h, i: (h * NB + i, 0)),
            pl.BlockSpec((BM, 1), lambda h, i: (i, 0)),
            pl.BlockSpec((1, 1, HALF), lambda h, i: (h, 0, 0)),
        ],
        out_specs=pl.BlockSpec((1, 2, HALF), lambda h, i: (h, 0, 0)),
        out_shape=jax.ShapeDtypeStruct((2, 2, HALF), F32),
    )(acc, dinv, b2)


def _apply_body(acc_ref, dinv_ref, b_ref, sc_ref, sh_ref, o_ref, *,
                fold_dinv):
    z = dinv_ref[...] * acc_ref[...] + b_ref[0]
    t = z * sc_ref[0] + sh_ref[0]
    t = jnp.maximum(t, 0.0)
    if fold_dinv:
        t = t * dinv_ref[...]
    o_ref[...] = t


def _tc_apply(acc, dinv, b2, scale2, shift2, fold_dinv):
    return pl.pallas_call(
        functools.partial(_apply_body, fold_dinv=fold_dinv),
        grid=(2, NB),
        in_specs=[
            pl.BlockSpec((BM, HALF), lambda h, i: (h * NB + i, 0)),
            pl.BlockSpec((BM, 1), lambda h, i: (i, 0)),
            pl.BlockSpec((1, 1, HALF), lambda h, i: (h, 0, 0)),
            pl.BlockSpec((1, 1, HALF), lambda h, i: (h, 0, 0)),
            pl.BlockSpec((1, 1, HALF), lambda h, i: (h, 0, 0)),
        ],
        out_specs=pl.BlockSpec((BM, HALF), lambda h, i: (i, h)),
        out_shape=jax.ShapeDtypeStruct((N, D_H), F32),
    )(acc, dinv, b2, scale2, shift2)


def _pool_body(x_ref, batch_ref, w0_ref, b0_ref, w1_ref, b1_ref, o_ref,
               acc_s, cnt_s):
    i = pl.program_id(0)

    @pl.when(i == 0)
    def _():
        acc_s[...] = jnp.zeros_like(acc_s)
        cnt_s[...] = jnp.zeros_like(cnt_s)

    iota = lax.broadcasted_iota(jnp.int32, (G, BM), 0)
    onehot = (iota == batch_ref[0]).astype(F32)
    acc_s[...] += jnp.dot(onehot, x_ref[...], preferred_element_type=F32,
                          precision=_HI)
    cnt_s[...] += jnp.sum(onehot, axis=1, keepdims=True)

    @pl.when(i == NB - 1)
    def _():
        h = acc_s[...] / jnp.maximum(cnt_s[...], 1.0)
        h1 = jnp.maximum(
            jnp.dot(h, w0_ref[...], preferred_element_type=F32,
                    precision=_HI) + b0_ref[...], 0.0)
        o_ref[...] = jnp.dot(h1, w1_ref[...], preferred_element_type=F32,
                             precision=_HI) + b1_ref[...]


def _tc_pool(x, batch2d, w0, b0, w1, b1):
    return pl.pallas_call(
        _pool_body,
        grid=(NB,),
        in_specs=[
            pl.BlockSpec((BM, D_H), lambda i: (i, 0)),
            pl.BlockSpec((1, 1, BM), lambda i: (i, 0, 0)),
            pl.BlockSpec((D_H, D_H), lambda i: (0, 0)),
            pl.BlockSpec((1, D_H), lambda i: (0, 0)),
            pl.BlockSpec((D_H, C), lambda i: (0, 0)),
            pl.BlockSpec((1, C), lambda i: (0, 0)),
        ],
        out_specs=pl.BlockSpec((G, C), lambda i: (0, 0)),
        out_shape=jax.ShapeDtypeStruct((G, C), F32),
        scratch_shapes=[
            pltpu.VMEM((G, D_H), F32),
            pltpu.VMEM((G, 1), F32),
        ],
    )(x, batch2d, w0, b0.reshape(1, D_H), w1, b1.reshape(1, C))


NCHUNK = E // CH

_DEG_CPT = NCHUNK // 2 // NT
_DEG_REM = NCHUNK // 2 - NT * _DEG_CPT


def _deg_body(col_hbm, out_hbm, cidx, ones_v, zbuf, acc_shared, sem):
    c = lax.axis_index("c")
    s = lax.axis_index("s")

    for k in range(CH // 16):
        ones_v[pl.ds(k * 16, 16)] = jnp.ones((16,), F32)
    for k in range(640 // 16):
        zbuf[pl.ds(k * 16, 16)] = jnp.zeros((16,), F32)

    start = s * 640
    pltpu.sync_copy(zbuf.at[pl.ds(0, 640)], acc_shared.at[pl.ds(start, 640)])

    plsc.subcore_barrier()

    base = c * (NCHUNK // 2) + s * _DEG_CPT

    def step(cid):
        pltpu.sync_copy(col_hbm.at[cid, 0], cidx)
        pltpu.sync_copy(ones_v, acc_shared.at[cidx], add=True)

    def body(j, carry):
        step(base + j)
        return carry

    lax.fori_loop(0, _DEG_CPT, body, 0, unroll=False)

    @pl.when(s < _DEG_REM)
    def _():
        step(c * (NCHUNK // 2) + NT * _DEG_CPT + s)

    plsc.subcore_barrier()

    pltpu.sync_copy(acc_shared.at[pl.ds(start, 640)],
                    out_hbm.at[c].at[pl.ds(start, 640)])


N_PAD = 10240

_sc_degree = pl.kernel(
    _deg_body,
    out_type=jax.ShapeDtypeStruct((2, N_PAD), F32),
    mesh=_SC_MESH,
    scratch_types=[
        pltpu.VMEM((CH,), jnp.int32),
        pltpu.VMEM((CH,), F32),
        pltpu.VMEM((640,), F32),
        pltpu.VMEM_SHARED((N_PAD,), F32),
        pltpu.SemaphoreType.DMA,
    ],
)


_INIT_RPT = 624
_PROP_CPT = NCHUNK // NT
_PROP_REM = NCHUNK - NT * _PROP_CPT


def _prop_body(ys_hbm, row_hbm, col_hbm, out_hbm, ridx_raw, ridx, cidx,
               rows, acc_shared, sem):
    c = lax.axis_index("c")
    s = lax.axis_index("s")
    c_n = c * N

    init = s * _INIT_RPT
    pltpu.sync_copy(ys_hbm.at[pl.ds(c_n + init, _INIT_RPT)],
                    acc_shared.at[pl.ds(init, _INIT_RPT)])

    @pl.when(s == NT - 1)
    def _():
        pltpu.sync_copy(ys_hbm.at[pl.ds(c_n + 9984, 16)],
                        acc_shared.at[pl.ds(9984, 16)])

    plsc.subcore_barrier()

    def step(cid):
        pltpu.sync_copy(row_hbm.at[cid, 0], ridx_raw)
        pltpu.sync_copy(col_hbm.at[cid, 0], cidx)
        for k in range(CH // 16):
            ridx[pl.ds(k * 16, 16)] = ridx_raw[pl.ds(k * 16, 16)] + c_n
        pltpu.async_copy(ys_hbm.at[ridx], rows, sem).wait()
        pltpu.sync_copy(rows, acc_shared.at[cidx], add=True)

    base = s * _PROP_CPT

    def body(j, carry):
        step(base + j)
        return carry

    lax.fori_loop(0, _PROP_CPT, body, 0, unroll=False)

    @pl.when(s < _PROP_REM)
    def _():
        step(NT * _PROP_CPT + s)

    plsc.subcore_barrier()

    pltpu.sync_copy(acc_shared.at[pl.ds(init, _INIT_RPT)],
                    out_hbm.at[pl.ds(c_n + init, _INIT_RPT)])

    @pl.when(s == NT - 1)
    def _():
        pltpu.sync_copy(acc_shared.at[pl.ds(9984, 16)],
                        out_hbm.at[pl.ds(c_n + 9984, 16)])


_sc_propagate = pl.kernel(
    _prop_body,
    out_type=jax.ShapeDtypeStruct((2 * N, HALF), F32),
    mesh=_SC_MESH,
    scratch_types=[
        pltpu.VMEM((CH,), jnp.int32),
        pltpu.VMEM((CH,), jnp.int32),
        pltpu.VMEM((CH,), jnp.int32),
        pltpu.VMEM((CH, HALF), F32),
        pltpu.VMEM_SHARED((N, HALF), F32),
        pltpu.SemaphoreType.DMA,
    ],
)


def kernel(pos, edge_index, batch, transform_W, transform_b,
           conv_W0, conv_b0, bn_g0, bn_b0,
           conv_W1, conv_b1, bn_g1, bn_b1,
           conv_W2, conv_b2, bn_g2, bn_b2,
           lin_W0, lin_b0, lin_W1, lin_b1):
    row = edge_index[0].reshape(NCHUNK, 1, CH)
    col = edge_index[1].reshape(NCHUNK, 1, CH)

    deg_parts = _sc_degree(col)
    x0 = _tc_transform(pos, transform_W, transform_b)

    deg = deg_parts[0, :N] + deg_parts[1, :N] + 1.0
    dinv = (deg ** -0.5).reshape(N, 1)

    layers = [(conv_W0, conv_b0, bn_g0, bn_b0),
              (conv_W1, conv_b1, bn_g1, bn_b1),
              (conv_W2, conv_b2, bn_g2, bn_b2)]

    x = x0
    out_feat = None
    for li, (w, b, g, be) in enumerate(layers):
        ys = _tc_pre(x, w, dinv if li == 0 else None)
        acc = _sc_propagate(ys, row, col)
        b2 = b.reshape(2, 1, HALF)
        sums = _tc_stats(acc, dinv, b2)
        mu = sums[:, :1, :] / N
        var = sums[:, 1:, :] / N - mu * mu
        scale2 = g.reshape(2, 1, HALF) * lax.rsqrt(var + 1e-5)
        shift2 = be.reshape(2, 1, HALF) - mu * scale2
        x = _tc_apply(acc, dinv, b2, scale2, shift2, fold_dinv=(li < 2))
        if li == 2:
            out_feat = x

    h = _tc_pool(out_feat, batch.reshape(NB, 1, BM), lin_W0, lin_b0,
                 lin_W1, lin_b1)
    return (h, out_feat)


# speedup vs baseline: 16.6206x; 1.8055x over previous
import functools

import jax
import jax.numpy as jnp
from jax import lax
from jax.experimental import pallas as pl
from jax.experimental.pallas import tpu as pltpu
from jax.experimental.pallas import tpu_sc as plsc

N = 10000
E = 320000
D_IN = 128
D_H = 256
HALF = 128
G = 64
C = 40

BM = 1000
NB = N // BM
NT = 16
CH = 128
EPT = E // NT
F32 = jnp.float32

_SC_MESH = plsc.VectorSubcoreMesh(core_axis_name="c", subcore_axis_name="s")
_HI = lax.Precision.HIGHEST


def _transform_body(x_ref, w_ref, b_ref, o_ref):
    o_ref[...] = (
        jnp.dot(x_ref[...], w_ref[...], preferred_element_type=F32,
                precision=_HI)
        + b_ref[...]
    )


def _tc_transform(pos, w, b):
    return pl.pallas_call(
        _transform_body,
        grid=(NB,),
        in_specs=[
            pl.BlockSpec((BM, D_IN), lambda i: (i, 0)),
            pl.BlockSpec((D_IN, D_IN), lambda i: (0, 0)),
            pl.BlockSpec((1, D_IN), lambda i: (0, 0)),
        ],
        out_specs=pl.BlockSpec((BM, D_IN), lambda i: (i, 0)),
        out_shape=jax.ShapeDtypeStruct((N, D_IN), F32),
    )(pos, w, b.reshape(1, D_IN))


def _pre_body_scaled(x_ref, w_ref, dinv_ref, o_ref):
    o_ref[...] = dinv_ref[...] * jnp.dot(
        x_ref[...], w_ref[...], preferred_element_type=F32, precision=_HI)


def _pre_body(x_ref, w_ref, o_ref):
    o_ref[...] = jnp.dot(
        x_ref[...], w_ref[...], preferred_element_type=F32, precision=_HI)


def _tc_pre(x, w, dinv=None):
    d = x.shape[1]
    if dinv is not None:
        body = _pre_body_scaled
        extra_specs = [pl.BlockSpec((BM, 1), lambda h, i: (i, 0))]
        extra_args = (dinv,)
    else:
        body = _pre_body
        extra_specs = []
        extra_args = ()
    return pl.pallas_call(
        body,
        grid=(2, NB),
        in_specs=[
            pl.BlockSpec((BM, d), lambda h, i: (i, 0)),
            pl.BlockSpec((d, HALF), lambda h, i: (0, h)),
        ] + extra_specs,
        out_specs=pl.BlockSpec((BM, HALF), lambda h, i: (h * NB + i, 0)),
        out_shape=jax.ShapeDtypeStruct((2 * N, HALF), F32),
    )(x, w, *extra_args)


def _stats_body(acc_ref, dinv_ref, b_ref, o_ref):
    i = pl.program_id(1)

    @pl.when(i == 0)
    def _():
        o_ref[...] = jnp.zeros_like(o_ref)

    z = dinv_ref[...] * acc_ref[...] + b_ref[0]
    s1 = jnp.sum(z, axis=0, keepdims=True)
    s2 = jnp.sum(z * z, axis=0, keepdims=True)
    o_ref[...] += jnp.stack([s1, s2], axis=1)


def _tc_stats(acc, dinv, b2):
    return pl.pallas_call(
        _stats_body,
        grid=(2, NB),
        in_specs=[
            pl.BlockSpec((BM, HALF), lambda h, i: (h * NB + i, 0)),
            pl.BlockSpec((BM, 1), lambda h, i: (i, 0)),
            pl.BlockSpec((1, 1, HALF), lambda h, i: (h, 0, 0)),
        ],
        out_specs=pl.BlockSpec((1, 2, HALF), lambda h, i: (h, 0, 0)),
        out_shape=jax.ShapeDtypeStruct((2, 2, HALF), F32),
    )(acc, dinv, b2)


def _apply_body(acc_ref, dinv_ref, b_ref, sc_ref, sh_ref, o_ref, *,
                fold_dinv):
    z = dinv_ref[...] * acc_ref[...] + b_ref[0]
    t = z * sc_ref[0] + sh_ref[0]
    t = jnp.maximum(t, 0.0)
    if fold_dinv:
        t = t * dinv_ref[...]
    o_ref[...] = t


def _tc_apply(acc, dinv, b2, scale2, shift2, fold_dinv):
    return pl.pallas_call(
        functools.partial(_apply_body, fold_dinv=fold_dinv),
        grid=(2, NB),
        in_specs=[
            pl.BlockSpec((BM, HALF), lambda h, i: (h * NB + i, 0)),
            pl.BlockSpec((BM, 1), lambda h, i: (i, 0)),
            pl.BlockSpec((1, 1, HALF), lambda h, i: (h, 0, 0)),
            pl.BlockSpec((1, 1, HALF), lambda h, i: (h, 0, 0)),
            pl.BlockSpec((1, 1, HALF), lambda h, i: (h, 0, 0)),
        ],
        out_specs=pl.BlockSpec((BM, HALF), lambda h, i: (i, h)),
        out_shape=jax.ShapeDtypeStruct((N, D_H), F32),
    )(acc, dinv, b2, scale2, shift2)


def _pool_body(x_ref, batch_ref, w0_ref, b0_ref, w1_ref, b1_ref, o_ref,
               acc_s, cnt_s):
    i = pl.program_id(0)

    @pl.when(i == 0)
    def _():
        acc_s[...] = jnp.zeros_like(acc_s)
        cnt_s[...] = jnp.zeros_like(cnt_s)

    iota = lax.broadcasted_iota(jnp.int32, (G, BM), 0)
    onehot = (iota == batch_ref[0]).astype(F32)
    acc_s[...] += jnp.dot(onehot, x_ref[...], preferred_element_type=F32,
                          precision=_HI)
    cnt_s[...] += jnp.sum(onehot, axis=1, keepdims=True)

    @pl.when(i == NB - 1)
    def _():
        h = acc_s[...] / jnp.maximum(cnt_s[...], 1.0)
        h1 = jnp.maximum(
            jnp.dot(h, w0_ref[...], preferred_element_type=F32,
                    precision=_HI) + b0_ref[...], 0.0)
        o_ref[...] = jnp.dot(h1, w1_ref[...], preferred_element_type=F32,
                             precision=_HI) + b1_ref[...]


def _tc_pool(x, batch2d, w0, b0, w1, b1):
    return pl.pallas_call(
        _pool_body,
        grid=(NB,),
        in_specs=[
            pl.BlockSpec((BM, D_H), lambda i: (i, 0)),
            pl.BlockSpec((1, 1, BM), lambda i: (i, 0, 0)),
            pl.BlockSpec((D_H, D_H), lambda i: (0, 0)),
            pl.BlockSpec((1, D_H), lambda i: (0, 0)),
            pl.BlockSpec((D_H, C), lambda i: (0, 0)),
            pl.BlockSpec((1, C), lambda i: (0, 0)),
        ],
        out_specs=pl.BlockSpec((G, C), lambda i: (0, 0)),
        out_shape=jax.ShapeDtypeStruct((G, C), F32),
        scratch_shapes=[
            pltpu.VMEM((G, D_H), F32),
            pltpu.VMEM((G, 1), F32),
        ],
    )(x, batch2d, w0, b0.reshape(1, D_H), w1, b1.reshape(1, C))


NCHUNK = E // CH

_DEG_CPT = NCHUNK // 2 // NT
_DEG_REM = NCHUNK // 2 - NT * _DEG_CPT


def _deg_body(col_hbm, out_hbm, cidx, ones_v, zbuf, acc_shared, sem):
    c = lax.axis_index("c")
    s = lax.axis_index("s")

    for k in range(CH // 16):
        ones_v[pl.ds(k * 16, 16)] = jnp.ones((16,), F32)
    for k in range(640 // 16):
        zbuf[pl.ds(k * 16, 16)] = jnp.zeros((16,), F32)

    start = s * 640
    pltpu.sync_copy(zbuf.at[pl.ds(0, 640)], acc_shared.at[pl.ds(start, 640)])

    plsc.subcore_barrier()

    base = c * (NCHUNK // 2) + s * _DEG_CPT

    def step(cid):
        pltpu.sync_copy(col_hbm.at[cid, 0], cidx)
        pltpu.sync_copy(ones_v, acc_shared.at[cidx], add=True)

    def body(j, carry):
        step(base + j)
        return carry

    lax.fori_loop(0, _DEG_CPT, body, 0, unroll=False)

    @pl.when(s < _DEG_REM)
    def _():
        step(c * (NCHUNK // 2) + NT * _DEG_CPT + s)

    plsc.subcore_barrier()

    pltpu.sync_copy(acc_shared.at[pl.ds(start, 640)],
                    out_hbm.at[c].at[pl.ds(start, 640)])


N_PAD = 10240

_sc_degree = pl.kernel(
    _deg_body,
    out_type=jax.ShapeDtypeStruct((2, N_PAD), F32),
    mesh=_SC_MESH,
    scratch_types=[
        pltpu.VMEM((CH,), jnp.int32),
        pltpu.VMEM((CH,), F32),
        pltpu.VMEM((640,), F32),
        pltpu.VMEM_SHARED((N_PAD,), F32),
        pltpu.SemaphoreType.DMA,
    ],
)


_INIT_RPT = 624
_PROP_CPT = NCHUNK // NT
_PROP_REM = NCHUNK - NT * _PROP_CPT


def _prop_body(ys_hbm, row_hbm, col_hbm, out_hbm, acc_shared, *,
               rbufs, cbufs, rowbufs, isems, gsems):
    c = lax.axis_index("c")
    s = lax.axis_index("s")
    c_n = c * N

    init = s * _INIT_RPT
    pltpu.sync_copy(ys_hbm.at[pl.ds(c_n + init, _INIT_RPT)],
                    acc_shared.at[pl.ds(init, _INIT_RPT)])

    @pl.when(s == NT - 1)
    def _():
        pltpu.sync_copy(ys_hbm.at[pl.ds(c_n + 9984, 16)],
                        acc_shared.at[pl.ds(9984, 16)])

    base = s * _PROP_CPT

    def load_idx(cid, p):
        pltpu.async_copy(row_hbm.at[cid, 0], rbufs[p], isems[p])
        pltpu.async_copy(col_hbm.at[cid, 0], cbufs[p], isems[p])

    def wait_idx(cid, p):
        pltpu.make_async_copy(row_hbm.at[cid, 0], rbufs[p], isems[p]).wait()
        pltpu.make_async_copy(col_hbm.at[cid, 0], cbufs[p], isems[p]).wait()

    def fire_gather(p):
        for k in range(CH // 16):
            rbufs[p][pl.ds(k * 16, 16)] = rbufs[p][pl.ds(k * 16, 16)] + c_n
        pltpu.async_copy(ys_hbm.at[rbufs[p]], rowbufs[p], gsems[p])

    def drain(p):
        pltpu.make_async_copy(ys_hbm.at[rbufs[p]], rowbufs[p],
                              gsems[p]).wait()
        pltpu.sync_copy(rowbufs[p], acc_shared.at[cbufs[p]], add=True)

    plsc.subcore_barrier()

    for p in range(3):
        load_idx(base + p, p)
    wait_idx(base, 0)
    fire_gather(0)
    wait_idx(base + 1, 1)
    fire_gather(1)

    def substep(j, p):
        @pl.when(j + 2 < _PROP_CPT)
        def _():
            wait_idx(base + j + 2, (p + 2) % 3)
            fire_gather((p + 2) % 3)

        drain(p)

        @pl.when(j + 3 < _PROP_CPT)
        def _():
            load_idx(base + j + 3, p)

    def body(i, carry):
        j = 3 * i
        substep(j, 0)
        substep(j + 1, 1)
        substep(j + 2, 2)
        return carry

    lax.fori_loop(0, _PROP_CPT // 3, body, 0, unroll=False)

    @pl.when(s < _PROP_REM)
    def _():
        xtra = NT * _PROP_CPT + s
        load_idx(xtra, 0)
        wait_idx(xtra, 0)
        fire_gather(0)
        drain(0)

    plsc.subcore_barrier()

    pltpu.sync_copy(acc_shared.at[pl.ds(init, _INIT_RPT)],
                    out_hbm.at[pl.ds(c_n + init, _INIT_RPT)])

    @pl.when(s == NT - 1)
    def _():
        pltpu.sync_copy(acc_shared.at[pl.ds(9984, 16)],
                        out_hbm.at[pl.ds(c_n + 9984, 16)])


def _prop_wrap(ys_hbm, row_hbm, col_hbm, out_hbm,
               rb0, rb1, rb2, cb0, cb1, cb2, rw0, rw1, rw2,
               acc_shared, is0, is1, is2, gs0, gs1, gs2):
    global_refs = dict(
        rbufs=[rb0, rb1, rb2], cbufs=[cb0, cb1, cb2],
        rowbufs=[rw0, rw1, rw2], isems=[is0, is1, is2],
        gsems=[gs0, gs1, gs2])
    _prop_body(ys_hbm, row_hbm, col_hbm, out_hbm, acc_shared,
               **global_refs)


_sc_propagate = pl.kernel(
    _prop_wrap,
    out_type=jax.ShapeDtypeStruct((2 * N, HALF), F32),
    mesh=_SC_MESH,
    scratch_types=[
        pltpu.VMEM((CH,), jnp.int32),
        pltpu.VMEM((CH,), jnp.int32),
        pltpu.VMEM((CH,), jnp.int32),
        pltpu.VMEM((CH,), jnp.int32),
        pltpu.VMEM((CH,), jnp.int32),
        pltpu.VMEM((CH,), jnp.int32),
        pltpu.VMEM((CH, HALF), F32),
        pltpu.VMEM((CH, HALF), F32),
        pltpu.VMEM((CH, HALF), F32),
        pltpu.VMEM_SHARED((N, HALF), F32),
        pltpu.SemaphoreType.DMA,
        pltpu.SemaphoreType.DMA,
        pltpu.SemaphoreType.DMA,
        pltpu.SemaphoreType.DMA,
        pltpu.SemaphoreType.DMA,
        pltpu.SemaphoreType.DMA,
    ],
)


def kernel(pos, edge_index, batch, transform_W, transform_b,
           conv_W0, conv_b0, bn_g0, bn_b0,
           conv_W1, conv_b1, bn_g1, bn_b1,
           conv_W2, conv_b2, bn_g2, bn_b2,
           lin_W0, lin_b0, lin_W1, lin_b1):
    row = edge_index[0].reshape(NCHUNK, 1, CH)
    col = edge_index[1].reshape(NCHUNK, 1, CH)

    deg_parts = _sc_degree(col)
    x0 = _tc_transform(pos, transform_W, transform_b)

    deg = deg_parts[0, :N] + deg_parts[1, :N] + 1.0
    dinv = (deg ** -0.5).reshape(N, 1)

    layers = [(conv_W0, conv_b0, bn_g0, bn_b0),
              (conv_W1, conv_b1, bn_g1, bn_b1),
              (conv_W2, conv_b2, bn_g2, bn_b2)]

    x = x0
    out_feat = None
    for li, (w, b, g, be) in enumerate(layers):
        ys = _tc_pre(x, w, dinv if li == 0 else None)
        acc = _sc_propagate(ys, row, col)
        b2 = b.reshape(2, 1, HALF)
        sums = _tc_stats(acc, dinv, b2)
        mu = sums[:, :1, :] / N
        var = sums[:, 1:, :] / N - mu * mu
        scale2 = g.reshape(2, 1, HALF) * lax.rsqrt(var + 1e-5)
        shift2 = be.reshape(2, 1, HALF) - mu * scale2
        x = _tc_apply(acc, dinv, b2, scale2, shift2, fold_dinv=(li < 2))
        if li == 2:
            out_feat = x

    h = _tc_pool(out_feat, batch.reshape(NB, 1, BM), lin_W0, lin_b0,
                 lin_W1, lin_b1)
    return (h, out_feat)


# speedup vs baseline: 20.4324x; 1.2293x over previous
import functools

import jax
import jax.numpy as jnp
from jax import lax
from jax.experimental import pallas as pl
from jax.experimental.pallas import tpu as pltpu
from jax.experimental.pallas import tpu_sc as plsc

N = 10000
E = 320000
D_IN = 128
D_H = 256
HALF = 128
G = 64
C = 40

BM = 1000
NB = N // BM
NT = 16
CH = 128
EPT = E // NT
F32 = jnp.float32

_SC_MESH = plsc.VectorSubcoreMesh(core_axis_name="c", subcore_axis_name="s")
_HI = lax.Precision.HIGHEST


def _transform_body(x_ref, w_ref, b_ref, o_ref):
    o_ref[...] = (
        jnp.dot(x_ref[...], w_ref[...], preferred_element_type=F32,
                precision=_HI)
        + b_ref[...]
    )


def _tc_transform(pos, w, b):
    return pl.pallas_call(
        _transform_body,
        grid=(NB,),
        in_specs=[
            pl.BlockSpec((BM, D_IN), lambda i: (i, 0)),
            pl.BlockSpec((D_IN, D_IN), lambda i: (0, 0)),
            pl.BlockSpec((1, D_IN), lambda i: (0, 0)),
        ],
        out_specs=pl.BlockSpec((BM, D_IN), lambda i: (i, 0)),
        out_shape=jax.ShapeDtypeStruct((N, D_IN), F32),
    )(pos, w, b.reshape(1, D_IN))


def _pre_body_scaled(x_ref, w_ref, dinv_ref, o_ref):
    o_ref[...] = dinv_ref[...] * jnp.dot(
        x_ref[...], w_ref[...], preferred_element_type=F32, precision=_HI)


def _pre_body(x_ref, w_ref, o_ref):
    o_ref[...] = jnp.dot(
        x_ref[...], w_ref[...], preferred_element_type=F32, precision=_HI)


def _tc_pre(x, w, dinv=None):
    d = x.shape[1]
    if dinv is not None:
        body = _pre_body_scaled
        extra_specs = [pl.BlockSpec((BM, 1), lambda h, i: (i, 0))]
        extra_args = (dinv,)
    else:
        body = _pre_body
        extra_specs = []
        extra_args = ()
    return pl.pallas_call(
        body,
        grid=(2, NB),
        in_specs=[
            pl.BlockSpec((BM, d), lambda h, i: (i, 0)),
            pl.BlockSpec((d, HALF), lambda h, i: (0, h)),
        ] + extra_specs,
        out_specs=pl.BlockSpec((BM, HALF), lambda h, i: (h * NB + i, 0)),
        out_shape=jax.ShapeDtypeStruct((2 * N, HALF), F32),
    )(x, w, *extra_args)


def _stats_body(acc_ref, dinv_ref, b_ref, o_ref):
    i = pl.program_id(1)

    @pl.when(i == 0)
    def _():
        o_ref[...] = jnp.zeros_like(o_ref)

    z = dinv_ref[...] * acc_ref[...] + b_ref[0]
    s1 = jnp.sum(z, axis=0, keepdims=True)
    s2 = jnp.sum(z * z, axis=0, keepdims=True)
    o_ref[...] += jnp.stack([s1, s2], axis=1)


def _tc_stats(acc, dinv, b2):
    return pl.pallas_call(
        _stats_body,
        grid=(2, NB),
        in_specs=[
            pl.BlockSpec((BM, HALF), lambda h, i: (h * NB + i, 0)),
            pl.BlockSpec((BM, 1), lambda h, i: (i, 0)),
            pl.BlockSpec((1, 1, HALF), lambda h, i: (h, 0, 0)),
        ],
        out_specs=pl.BlockSpec((1, 2, HALF), lambda h, i: (h, 0, 0)),
        out_shape=jax.ShapeDtypeStruct((2, 2, HALF), F32),
    )(acc, dinv, b2)


def _apply_body(acc_ref, dinv_ref, b_ref, sc_ref, sh_ref, o_ref, *,
                fold_dinv):
    z = dinv_ref[...] * acc_ref[...] + b_ref[0]
    t = z * sc_ref[0] + sh_ref[0]
    t = jnp.maximum(t, 0.0)
    if fold_dinv:
        t = t * dinv_ref[...]
    o_ref[...] = t


def _tc_apply(acc, dinv, b2, scale2, shift2, fold_dinv):
    return pl.pallas_call(
        functools.partial(_apply_body, fold_dinv=fold_dinv),
        grid=(2, NB),
        in_specs=[
            pl.BlockSpec((BM, HALF), lambda h, i: (h * NB + i, 0)),
            pl.BlockSpec((BM, 1), lambda h, i: (i, 0)),
            pl.BlockSpec((1, 1, HALF), lambda h, i: (h, 0, 0)),
            pl.BlockSpec((1, 1, HALF), lambda h, i: (h, 0, 0)),
            pl.BlockSpec((1, 1, HALF), lambda h, i: (h, 0, 0)),
        ],
        out_specs=pl.BlockSpec((BM, HALF), lambda h, i: (i, h)),
        out_shape=jax.ShapeDtypeStruct((N, D_H), F32),
    )(acc, dinv, b2, scale2, shift2)


def _fused_body(acc0_ref, acc1_ref, dinv_ref, b_ref, sc_ref, sh_ref, w_ref,
                o_ref):
    dinv = dinv_ref[...]
    t0 = jnp.maximum((dinv * acc0_ref[...] + b_ref[0, 0]) * sc_ref[0, 0]
                     + sh_ref[0, 0], 0.0) * dinv
    t1 = jnp.maximum((dinv * acc1_ref[...] + b_ref[1, 0]) * sc_ref[1, 0]
                     + sh_ref[1, 0], 0.0) * dinv
    x_blk = jnp.concatenate([t0, t1], axis=1)
    o_ref[...] = jnp.dot(x_blk, w_ref[...], preferred_element_type=F32,
                         precision=_HI)


def _tc_fused(acc, dinv, b2, scale2, shift2, w_next):
    return pl.pallas_call(
        _fused_body,
        grid=(2, NB),
        in_specs=[
            pl.BlockSpec((BM, HALF), lambda h, i: (i, 0)),
            pl.BlockSpec((BM, HALF), lambda h, i: (NB + i, 0)),
            pl.BlockSpec((BM, 1), lambda h, i: (i, 0)),
            pl.BlockSpec((2, 1, HALF), lambda h, i: (0, 0, 0)),
            pl.BlockSpec((2, 1, HALF), lambda h, i: (0, 0, 0)),
            pl.BlockSpec((2, 1, HALF), lambda h, i: (0, 0, 0)),
            pl.BlockSpec((D_H, HALF), lambda h, i: (0, h)),
        ],
        out_specs=pl.BlockSpec((BM, HALF), lambda h, i: (h * NB + i, 0)),
        out_shape=jax.ShapeDtypeStruct((2 * N, HALF), F32),
    )(acc, acc, dinv, b2, scale2, shift2, w_next)


def _pool_body(x_ref, batch_ref, w0_ref, b0_ref, w1_ref, b1_ref, o_ref,
               acc_s, cnt_s):
    i = pl.program_id(0)

    @pl.when(i == 0)
    def _():
        acc_s[...] = jnp.zeros_like(acc_s)
        cnt_s[...] = jnp.zeros_like(cnt_s)

    iota = lax.broadcasted_iota(jnp.int32, (G, BM), 0)
    onehot = (iota == batch_ref[0]).astype(F32)
    acc_s[...] += jnp.dot(onehot, x_ref[...], preferred_element_type=F32,
                          precision=_HI)
    cnt_s[...] += jnp.sum(onehot, axis=1, keepdims=True)

    @pl.when(i == NB - 1)
    def _():
        h = acc_s[...] / jnp.maximum(cnt_s[...], 1.0)
        h1 = jnp.maximum(
            jnp.dot(h, w0_ref[...], preferred_element_type=F32,
                    precision=_HI) + b0_ref[...], 0.0)
        o_ref[...] = jnp.dot(h1, w1_ref[...], preferred_element_type=F32,
                             precision=_HI) + b1_ref[...]


def _tc_pool(x, batch2d, w0, b0, w1, b1):
    return pl.pallas_call(
        _pool_body,
        grid=(NB,),
        in_specs=[
            pl.BlockSpec((BM, D_H), lambda i: (i, 0)),
            pl.BlockSpec((1, 1, BM), lambda i: (i, 0, 0)),
            pl.BlockSpec((D_H, D_H), lambda i: (0, 0)),
            pl.BlockSpec((1, D_H), lambda i: (0, 0)),
            pl.BlockSpec((D_H, C), lambda i: (0, 0)),
            pl.BlockSpec((1, C), lambda i: (0, 0)),
        ],
        out_specs=pl.BlockSpec((G, C), lambda i: (0, 0)),
        out_shape=jax.ShapeDtypeStruct((G, C), F32),
        scratch_shapes=[
            pltpu.VMEM((G, D_H), F32),
            pltpu.VMEM((G, 1), F32),
        ],
    )(x, batch2d, w0, b0.reshape(1, D_H), w1, b1.reshape(1, C))


NCHUNK = E // CH

_DEG_CPT = NCHUNK // 2 // NT
_DEG_REM = NCHUNK // 2 - NT * _DEG_CPT


def _deg_body(col_hbm, out_hbm, cidx, ones_v, zbuf, acc_shared, sem):
    c = lax.axis_index("c")
    s = lax.axis_index("s")

    for k in range(CH // 16):
        ones_v[pl.ds(k * 16, 16)] = jnp.ones((16,), F32)
    for k in range(640 // 16):
        zbuf[pl.ds(k * 16, 16)] = jnp.zeros((16,), F32)

    start = s * 640
    pltpu.sync_copy(zbuf.at[pl.ds(0, 640)], acc_shared.at[pl.ds(start, 640)])

    plsc.subcore_barrier()

    base = c * (NCHUNK // 2) + s * _DEG_CPT

    def step(cid):
        pltpu.sync_copy(col_hbm.at[cid, 0], cidx)
        pltpu.sync_copy(ones_v, acc_shared.at[cidx], add=True)

    def body(j, carry):
        step(base + j)
        return carry

    lax.fori_loop(0, _DEG_CPT, body, 0, unroll=False)

    @pl.when(s < _DEG_REM)
    def _():
        step(c * (NCHUNK // 2) + NT * _DEG_CPT + s)

    plsc.subcore_barrier()

    pltpu.sync_copy(acc_shared.at[pl.ds(start, 640)],
                    out_hbm.at[c].at[pl.ds(start, 640)])


N_PAD = 10240

_sc_degree = pl.kernel(
    _deg_body,
    out_type=jax.ShapeDtypeStruct((2, N_PAD), F32),
    mesh=_SC_MESH,
    scratch_types=[
        pltpu.VMEM((CH,), jnp.int32),
        pltpu.VMEM((CH,), F32),
        pltpu.VMEM((640,), F32),
        pltpu.VMEM_SHARED((N_PAD,), F32),
        pltpu.SemaphoreType.DMA,
    ],
)


_INIT_RPT = 624
_PROP_CPT = NCHUNK // NT
_PROP_REM = NCHUNK - NT * _PROP_CPT


def _prop_body(ys_hbm, row_hbm, col_hbm, out_hbm, acc_shared, *,
               rbufs, cbufs, rowbufs, isems, gsems, ssems):
    c = lax.axis_index("c")
    s = lax.axis_index("s")
    c_n = c * N

    init = s * _INIT_RPT
    pltpu.sync_copy(ys_hbm.at[pl.ds(c_n + init, _INIT_RPT)],
                    acc_shared.at[pl.ds(init, _INIT_RPT)])

    @pl.when(s == NT - 1)
    def _():
        pltpu.sync_copy(ys_hbm.at[pl.ds(c_n + 9984, 16)],
                        acc_shared.at[pl.ds(9984, 16)])

    base = s * _PROP_CPT

    def load_idx(cid, q):
        pltpu.async_copy(row_hbm.at[cid, 0], rbufs[q], isems[q])
        pltpu.async_copy(col_hbm.at[cid, 0], cbufs[q], isems[q])

    def wait_idx(cid, q):
        pltpu.make_async_copy(row_hbm.at[cid, 0], rbufs[q], isems[q]).wait()
        pltpu.make_async_copy(col_hbm.at[cid, 0], cbufs[q], isems[q]).wait()

    def fire_gather(q, p):
        for k in range(CH // 16):
            rbufs[q][pl.ds(k * 16, 16)] = rbufs[q][pl.ds(k * 16, 16)] + c_n
        pltpu.async_copy(ys_hbm.at[rbufs[q]], rowbufs[p], gsems[p])

    def wait_gather(q, p):
        pltpu.make_async_copy(ys_hbm.at[rbufs[q]], rowbufs[p],
                              gsems[p]).wait()

    def fire_scatter(q, p):
        pltpu.async_copy(rowbufs[p], acc_shared.at[cbufs[q]], ssems[p],
                         add=True)

    def wait_scatter(p):
        pltpu.make_async_copy(rowbufs[p], acc_shared.at[cbufs[0]],
                              ssems[p]).wait()

    plsc.subcore_barrier()

    for q in range(3):
        load_idx(base + q, q)
    wait_idx(base, 0)
    fire_gather(0, 0)
    wait_idx(base + 1, 1)
    fire_gather(1, 1)

    def substep(j, q):
        p = q % 3

        @pl.when(j + 2 < _PROP_CPT)
        def _():
            q2 = (q + 2) % 6
            p2 = (q + 2) % 3
            wait_idx(base + j + 2, q2)

            @pl.when(j >= 1)
            def _():
                wait_scatter(p2)

            fire_gather(q2, p2)

        wait_gather(q, p)
        fire_scatter(q, p)

        @pl.when(j + 3 < _PROP_CPT)
        def _():
            load_idx(base + j + 3, (q + 3) % 6)

    def body(i, carry):
        j = 6 * i
        for t in range(6):
            substep(j + t, t)
        return carry

    lax.fori_loop(0, _PROP_CPT // 6, body, 0, unroll=False)

    wait_scatter(0)
    wait_scatter(1)
    wait_scatter(2)

    @pl.when(s < _PROP_REM)
    def _():
        xtra = NT * _PROP_CPT + s
        load_idx(xtra, 0)
        wait_idx(xtra, 0)
        fire_gather(0, 0)
        wait_gather(0, 0)
        pltpu.sync_copy(rowbufs[0], acc_shared.at[cbufs[0]], add=True)

    plsc.subcore_barrier()

    pltpu.sync_copy(acc_shared.at[pl.ds(init, _INIT_RPT)],
                    out_hbm.at[pl.ds(c_n + init, _INIT_RPT)])

    @pl.when(s == NT - 1)
    def _():
        pltpu.sync_copy(acc_shared.at[pl.ds(9984, 16)],
                        out_hbm.at[pl.ds(c_n + 9984, 16)])


def _prop_wrap(ys_hbm, row_hbm, col_hbm, out_hbm,
               rb0, rb1, rb2, rb3, rb4, rb5,
               cb0, cb1, cb2, cb3, cb4, cb5,
               rw0, rw1, rw2,
               acc_shared,
               is0, is1, is2, is3, is4, is5,
               gs0, gs1, gs2, ss0, ss1, ss2):
    global_refs = dict(
        rbufs=[rb0, rb1, rb2, rb3, rb4, rb5],
        cbufs=[cb0, cb1, cb2, cb3, cb4, cb5],
        rowbufs=[rw0, rw1, rw2],
        isems=[is0, is1, is2, is3, is4, is5],
        gsems=[gs0, gs1, gs2],
        ssems=[ss0, ss1, ss2])
    _prop_body(ys_hbm, row_hbm, col_hbm, out_hbm, acc_shared,
               **global_refs)


_sc_propagate = pl.kernel(
    _prop_wrap,
    out_type=jax.ShapeDtypeStruct((2 * N, HALF), F32),
    mesh=_SC_MESH,
    scratch_types=(
        [pltpu.VMEM((CH,), jnp.int32)] * 12
        + [pltpu.VMEM((CH, HALF), F32)] * 3
        + [pltpu.VMEM_SHARED((N, HALF), F32)]
        + [pltpu.SemaphoreType.DMA] * 12
    ),
)


def kernel(pos, edge_index, batch, transform_W, transform_b,
           conv_W0, conv_b0, bn_g0, bn_b0,
           conv_W1, conv_b1, bn_g1, bn_b1,
           conv_W2, conv_b2, bn_g2, bn_b2,
           lin_W0, lin_b0, lin_W1, lin_b1):
    row = edge_index[0].reshape(NCHUNK, 1, CH)
    col = edge_index[1].reshape(NCHUNK, 1, CH)

    deg_parts = _sc_degree(col)
    x0 = _tc_transform(pos, transform_W, transform_b)

    deg = deg_parts[0, :N] + deg_parts[1, :N] + 1.0
    dinv = (deg ** -0.5).reshape(N, 1)

    layers = [(conv_W0, conv_b0, bn_g0, bn_b0),
              (conv_W1, conv_b1, bn_g1, bn_b1),
              (conv_W2, conv_b2, bn_g2, bn_b2)]

    ys = _tc_pre(x0, conv_W0, dinv)
    out_feat = None
    for li, (w, b, g, be) in enumerate(layers):
        acc = _sc_propagate(ys, row, col)
        b2 = b.reshape(2, 1, HALF)
        sums = _tc_stats(acc, dinv, b2)
        mu = sums[:, :1, :] / N
        var = sums[:, 1:, :] / N - mu * mu
        scale2 = g.reshape(2, 1, HALF) * lax.rsqrt(var + 1e-5)
        shift2 = be.reshape(2, 1, HALF) - mu * scale2
        if li < 2:
            w_next = layers[li + 1][0]
            ys = _tc_fused(acc, dinv, b2, scale2, shift2, w_next)
        else:
            out_feat = _tc_apply(acc, dinv, b2, scale2, shift2,
                                 fold_dinv=False)

    h = _tc_pool(out_feat, batch.reshape(NB, 1, BM), lin_W0, lin_b0,
                 lin_W1, lin_b1)
    return (h, out_feat)
